# Initial kernel scaffold; baseline (speedup 1.0000x reference)
import jax
import jax.numpy as jnp
from jax.experimental import pallas as pl


def kernel(h, v, midpoint_pos, midpoint_theta, senders, receivers, gW1, gb1, gW2, gb2, gW3, gb3, pW1, pb1, pW2, pb2, pW3, pb3, sW1, sb1, sW2, sb2, sW3, sb3):
    raise NotImplementedError("write your pallas kernel here")


# speedup vs baseline: 20.3466x; 20.3466x over previous
import functools
import jax
import jax.numpy as jnp
from jax import lax
from jax.experimental import pallas as pl
from jax.experimental.pallas import tpu as pltpu
from jax.experimental.pallas import tpu_sc as plsc

N = 100000
E = 1600000
SD = 32
VD = 8
HD = 64

NPAD = 100352
EPAD = 1605632
BN = 1024
BE = 2048
NG = EPAD // 128
ROWS_T = NPAD // 16


def _silu(x):
    return x * jax.nn.sigmoid(x)


def _stage_a_body(h, vx, vy, px, py, th, pW1, pb1, pW2, pb2, pW3, pb3,
                  ts_ref, tr_ref):
    hb = h[...]
    x = _silu(jnp.dot(hb, pW1[...], preferred_element_type=jnp.float32) + pb1[...])
    x = _silu(jnp.dot(x, pW2[...], preferred_element_type=jnp.float32) + pb2[...])
    psi = jnp.dot(x, pW3[...], preferred_element_type=jnp.float32) + pb3[...]
    c = jnp.cos(th[...])
    s = jnp.sin(th[...])
    vxb = vx[...]
    vyb = vy[...]
    ts_ref[:, 0:32] = hb
    ts_ref[:, 32:40] = c * vxb - s * vyb
    ts_ref[:, 40:48] = s * vxb + c * vyb
    ts_ref[:, 48:56] = psi
    ts_ref[:, 56:64] = jnp.sqrt(vxb * vxb + vyb * vyb)
    ts_ref[:, 64:65] = px[...]
    ts_ref[:, 65:66] = py[...]
    ts_ref[:, 66:67] = c
    ts_ref[:, 67:68] = s
    ts_ref[:, 68:80] = jnp.zeros_like(ts_ref[:, 68:80])
    tr_ref[:, 0:32] = hb
    tr_ref[:, 32:33] = px[...]
    tr_ref[:, 33:34] = py[...]
    tr_ref[:, 34:35] = c
    tr_ref[:, 35:36] = s
    tr_ref[:, 36:48] = jnp.zeros_like(tr_ref[:, 36:48])


def _stage_a(h, vx, vy, px, py, th, pW1, pb1, pW2, pb2, pW3, pb3):
    nblk = NPAD // BN
    row = lambda i: (i, 0)
    fix = lambda i: (0, 0)
    return pl.pallas_call(
        _stage_a_body,
        grid=(nblk,),
        in_specs=[
            pl.BlockSpec((BN, SD), row),
            pl.BlockSpec((BN, VD), row),
            pl.BlockSpec((BN, VD), row),
            pl.BlockSpec((BN, 1), row),
            pl.BlockSpec((BN, 1), row),
            pl.BlockSpec((BN, 1), row),
            pl.BlockSpec((SD, HD), fix),
            pl.BlockSpec((1, HD), fix),
            pl.BlockSpec((HD, HD), fix),
            pl.BlockSpec((1, HD), fix),
            pl.BlockSpec((HD, VD), fix),
            pl.BlockSpec((1, VD), fix),
        ],
        out_specs=[
            pl.BlockSpec((BN, 80), row),
            pl.BlockSpec((BN, 48), row),
        ],
        out_shape=[
            jax.ShapeDtypeStruct((NPAD, 80), jnp.float32),
            jax.ShapeDtypeStruct((NPAD, 48), jnp.float32),
        ],
    )(h, vx, vy, px, py, th, pW1, pb1, pW2, pb2, pW3, pb3)


def _stage_b_body(ts_hbm, tr_hbm, sidx_hbm, ridx_hbm, gs_hbm, gr_hbm,
                  sidx_v, ridx_v, gs_v, gr_v):
    cid = lax.axis_index("c")
    sid = lax.axis_index("s")
    wid = cid * 16 + sid
    nbatch = NG // 128
    per_w = NG // (4 * 32)

    def body(i, _):
        b0 = wid * per_w + i
        g0 = b0 * 4
        pltpu.sync_copy(sidx_hbm.at[pl.ds(g0, 4)], sidx_v)
        pltpu.sync_copy(ridx_hbm.at[pl.ds(g0, 4)], ridx_v)
        for j in range(4):
            pltpu.sync_copy(ts_hbm.at[sidx_v.at[j]], gs_v.at[pl.ds(j * 128, 128)])
            pltpu.sync_copy(tr_hbm.at[ridx_v.at[j]], gr_v.at[pl.ds(j * 128, 128)])
        e0 = g0 * 128
        pltpu.sync_copy(gs_v, gs_hbm.at[pl.ds(e0, 512)])
        pltpu.sync_copy(gr_v, gr_hbm.at[pl.ds(e0, 512)])
        return ()

    lax.fori_loop(0, per_w, body, (), unroll=False)


def _stage_b(ts, tr, sidx2d, ridx2d):
    mesh = plsc.VectorSubcoreMesh(core_axis_name="c", subcore_axis_name="s")
    f = pl.kernel(
        _stage_b_body,
        out_type=[
            jax.ShapeDtypeStruct((EPAD, 80), jnp.float32),
            jax.ShapeDtypeStruct((EPAD, 48), jnp.float32),
        ],
        mesh=mesh,
        scratch_types=[
            pltpu.VMEM((4, 128), jnp.int32),
            pltpu.VMEM((4, 128), jnp.int32),
            pltpu.VMEM((512, 80), jnp.float32),
            pltpu.VMEM((512, 48), jnp.float32),
        ],
        compiler_params=pltpu.CompilerParams(use_tc_tiling_on_sc=False),
    )
    return f(ts, tr, sidx2d, ridx2d)


def _stage_c_body(gs, gr, W1s, W1r, Wr, Wc, Ws, Wvn, Wvd, b1,
                  W2, b2, W3, b3, msga_ref, msgb_ref, msgc_ref):
    gsb = gs[...]
    grb = gr[...]
    hs = gsb[:, 0:32]
    wsx = gsb[:, 32:40]
    wsy = gsb[:, 40:48]
    psis = gsb[:, 48:56]
    vns = gsb[:, 56:64]
    hr = grb[:, 0:32]
    dx = gsb[:, 64:65] - grb[:, 32:33]
    dy = gsb[:, 65:66] - grb[:, 33:34]
    r = jnp.sqrt(dx * dx + dy * dy)
    inv = 1.0 / (r + 1e-8)
    ux = dx * inv
    uy = dy * inv
    cs = gsb[:, 66:67]
    ss = gsb[:, 67:68]
    cr = grb[:, 34:35]
    sr = grb[:, 35:36]
    cd = cs * cr + ss * sr
    sd = ss * cr - cs * sr
    vd = wsx * ux + wsy * uy
    x1 = (jnp.dot(hs, W1s[...], preferred_element_type=jnp.float32)
          + jnp.dot(hr, W1r[...], preferred_element_type=jnp.float32))
    pre1 = (x1 + r * Wr[...] + cd * Wc[...] + sd * Ws[...]
            + jnp.dot(vns, Wvn[...], preferred_element_type=jnp.float32)
            + jnp.dot(vd, Wvd[...], preferred_element_type=jnp.float32)
            + b1[...])
    h1 = _silu(pre1)
    h2 = _silu(jnp.dot(h1, W2[...], preferred_element_type=jnp.float32) + b2[...])
    o3 = jnp.dot(h2, W3[...], preferred_element_type=jnp.float32) + b3[...]
    a = o3[:, 0:8]
    b_ = o3[:, 8:16]
    c_ = o3[:, 16:24]
    coef = b_ + c_ * psis
    msga_ref[...] = o3[:, 24:40]
    msgb_ref[...] = o3[:, 40:56]
    msgc_ref[:, 0:8] = a * wsx + coef * ux
    msgc_ref[:, 8:16] = a * wsy + coef * uy


def _stage_c(gs, gr, W1s, W1r, Wr, Wc, Ws, Wvn, Wvd, b1, W2, b2, W3, b3):
    nblk = EPAD // BE
    row = lambda i: (i, 0)
    fix = lambda i: (0, 0)
    return pl.pallas_call(
        _stage_c_body,
        grid=(nblk,),
        in_specs=[
            pl.BlockSpec((BE, 80), row),
            pl.BlockSpec((BE, 48), row),
            pl.BlockSpec((SD, 128), fix),
            pl.BlockSpec((SD, 128), fix),
            pl.BlockSpec((1, 128), fix),
            pl.BlockSpec((1, 128), fix),
            pl.BlockSpec((1, 128), fix),
            pl.BlockSpec((VD, 128), fix),
            pl.BlockSpec((VD, 128), fix),
            pl.BlockSpec((1, 128), fix),
            pl.BlockSpec((128, 128), fix),
            pl.BlockSpec((1, 128), fix),
            pl.BlockSpec((128, 56), fix),
            pl.BlockSpec((1, 56), fix),
        ],
        out_specs=[
            pl.BlockSpec((BE, 16), row),
            pl.BlockSpec((BE, 16), row),
            pl.BlockSpec((BE, 16), row),
        ],
        out_shape=[
            jax.ShapeDtypeStruct((EPAD, 16), jnp.float32),
            jax.ShapeDtypeStruct((EPAD, 16), jnp.float32),
            jax.ShapeDtypeStruct((EPAD, 16), jnp.float32),
        ],
    )(gs, gr, W1s, W1r, Wr, Wc, Ws, Wvn, Wvd, b1, W2, b2, W3, b3)


def _scatter_pass(msg_hbm, ridx_hbm, zer_hbm, out_hbm, acc, ibuf, vbuf,
                  sid, g_base, groups_per_tile):
    rows = pl.ds(sid * ROWS_T, ROWS_T)
    pltpu.sync_copy(zer_hbm.at[rows], acc.at[rows])
    plsc.subcore_barrier()
    nbatch = groups_per_tile // 4

    def body(i, _):
        g0 = g_base + sid * groups_per_tile + i * 4
        pltpu.sync_copy(ridx_hbm.at[pl.ds(g0, 4)], ibuf)
        pltpu.sync_copy(msg_hbm.at[pl.ds(g0 * 128, 512)], vbuf)
        for j in range(4):
            pltpu.sync_copy(vbuf.at[pl.ds(j * 128, 128)],
                            acc.at[ibuf.at[j]], add=True)
        return ()

    lax.fori_loop(0, nbatch, body, (), unroll=False)
    plsc.subcore_barrier()
    pltpu.sync_copy(acc.at[rows], out_hbm.at[rows])
    plsc.subcore_barrier()


def _stage_d_body(msga_hbm, msgb_hbm, msgc_hbm, ridx_hbm, zer_hbm,
                  a0_hbm, a1_hbm, a2a_hbm, a2b_hbm, acc, ibuf, vbuf):
    cid = lax.axis_index("c")
    sid = lax.axis_index("s")
    half = NG // 2

    @pl.when(cid == 0)
    def _():
        _scatter_pass(msga_hbm, ridx_hbm, zer_hbm, a0_hbm, acc, ibuf, vbuf,
                      sid, 0, NG // 16)
        _scatter_pass(msgc_hbm, ridx_hbm, zer_hbm, a2a_hbm, acc, ibuf, vbuf,
                      sid, 0, half // 16)

    @pl.when(cid == 1)
    def _():
        _scatter_pass(msgb_hbm, ridx_hbm, zer_hbm, a1_hbm, acc, ibuf, vbuf,
                      sid, 0, NG // 16)
        _scatter_pass(msgc_hbm, ridx_hbm, zer_hbm, a2b_hbm, acc, ibuf, vbuf,
                      sid, half, half // 16)


def _stage_d(msga, msgb, msgc, ridx2d, zer):
    mesh = plsc.VectorSubcoreMesh(core_axis_name="c", subcore_axis_name="s")
    f = pl.kernel(
        _stage_d_body,
        out_type=[
            jax.ShapeDtypeStruct((NPAD, 16), jnp.float32),
            jax.ShapeDtypeStruct((NPAD, 16), jnp.float32),
            jax.ShapeDtypeStruct((NPAD, 16), jnp.float32),
            jax.ShapeDtypeStruct((NPAD, 16), jnp.float32),
        ],
        mesh=mesh,
        scratch_types=[
            pltpu.VMEM_SHARED((NPAD, 16), jnp.float32),
            pltpu.VMEM((4, 128), jnp.int32),
            pltpu.VMEM((512, 16), jnp.float32),
        ],
        compiler_params=pltpu.CompilerParams(use_tc_tiling_on_sc=False),
    )
    return f(msga, msgb, msgc, ridx2d, zer)


def _stage_e_body(h, vx, vy, th, a0, a1, a2a, a2b, hn_ref, vxn_ref, vyn_ref):
    hn_ref[:, 0:16] = h[:, 0:16] + a0[...]
    hn_ref[:, 16:32] = h[:, 16:32] + a1[...]
    m = a2a[...] + a2b[...]
    mx = m[:, 0:8]
    my = m[:, 8:16]
    c = jnp.cos(th[...])
    s = jnp.sin(th[...])
    vxn_ref[...] = vx[...] + c * mx + s * my
    vyn_ref[...] = vy[...] - s * mx + c * my


def _stage_e(h, vx, vy, th, a0, a1, a2a, a2b):
    nblk = NPAD // BN
    row = lambda i: (i, 0)
    return pl.pallas_call(
        _stage_e_body,
        grid=(nblk,),
        in_specs=[
            pl.BlockSpec((BN, SD), row),
            pl.BlockSpec((BN, VD), row),
            pl.BlockSpec((BN, VD), row),
            pl.BlockSpec((BN, 1), row),
            pl.BlockSpec((BN, 16), row),
            pl.BlockSpec((BN, 16), row),
            pl.BlockSpec((BN, 16), row),
            pl.BlockSpec((BN, 16), row),
        ],
        out_specs=[
            pl.BlockSpec((BN, SD), row),
            pl.BlockSpec((BN, VD), row),
            pl.BlockSpec((BN, VD), row),
        ],
        out_shape=[
            jax.ShapeDtypeStruct((NPAD, SD), jnp.float32),
            jax.ShapeDtypeStruct((NPAD, VD), jnp.float32),
            jax.ShapeDtypeStruct((NPAD, VD), jnp.float32),
        ],
    )(h, vx, vy, th, a0, a1, a2a, a2b)


def kernel(h, v, midpoint_pos, midpoint_theta, senders, receivers,
           gW1, gb1, gW2, gb2, gW3, gb3, pW1, pb1, pW2, pb2, pW3, pb3,
           sW1, sb1, sW2, sb2, sW3, sb3):
    f32 = jnp.float32
    padn = NPAD - N
    hp = jnp.pad(h, ((0, padn), (0, 0)))
    vxp = jnp.pad(v[:, :, 0], ((0, padn), (0, 0)))
    vyp = jnp.pad(v[:, :, 1], ((0, padn), (0, 0)))
    pxp = jnp.pad(midpoint_pos[:, 0:1], ((0, padn), (0, 0)))
    pyp = jnp.pad(midpoint_pos[:, 1:2], ((0, padn), (0, 0)))
    thp = jnp.pad(midpoint_theta[:, None], ((0, padn), (0, 0)))

    pade = EPAD - E
    sidx = jnp.pad(senders.astype(jnp.int32), (0, pade)).reshape(NG, 128)
    ridx = jnp.pad(receivers.astype(jnp.int32), (0, pade),
                   constant_values=N).reshape(NG, 128)

    W1s = jnp.concatenate([gW1[0:32], sW1[0:32]], axis=1)
    W1r = jnp.concatenate([gW1[32:64], sW1[32:64]], axis=1)
    Wr = jnp.concatenate([gW1[64:65], sW1[64:65]], axis=1)
    Wc = jnp.concatenate([gW1[65:66], sW1[65:66]], axis=1)
    Ws = jnp.concatenate([gW1[66:67], sW1[66:67]], axis=1)
    z864 = jnp.zeros((8, 64), f32)
    Wvn = jnp.concatenate([z864, sW1[67:75]], axis=1)
    Wvd = jnp.concatenate([z864, sW1[75:83]], axis=1)
    b1c = jnp.concatenate([gb1, sb1])[None, :]
    W2b = jnp.zeros((128, 128), f32).at[0:64, 0:64].set(gW2).at[64:128, 64:128].set(sW2)
    b2c = jnp.concatenate([gb2, sb2])[None, :]
    W3b = jnp.zeros((128, 56), f32).at[0:64, 0:24].set(gW3).at[64:128, 24:56].set(sW3)
    b3c = jnp.concatenate([gb3, sb3])[None, :]

    ts, tr = _stage_a(hp, vxp, vyp, pxp, pyp, thp,
                      pW1, pb1[None, :], pW2, pb2[None, :], pW3, pb3[None, :])
    gs, gr = _stage_b(ts, tr, sidx, ridx)
    msga, msgb, msgc = _stage_c(gs, gr, W1s, W1r, Wr, Wc, Ws, Wvn, Wvd,
                                b1c, W2b, b2c, W3b, b3c)
    zer = jnp.zeros((NPAD, 16), f32)
    a0, a1, a2a, a2b = _stage_d(msga, msgb, msgc, ridx, zer)
    hn, vxn, vyn = _stage_e(hp, vxp, vyp, thp, a0, a1, a2a, a2b)
    h_new = hn[:N]
    v_new = jnp.stack([vxn[:N], vyn[:N]], axis=-1)
    return (h_new, v_new)
